# baseline (speedup 1.0000x reference)
import functools

import jax
import jax.numpy as jnp
from jax import lax
from jax.experimental import pallas as pl
from jax.experimental.pallas import tpu as pltpu


def _lrelu(x):
    return jnp.maximum(x, 0.2 * x)


def _rep(shape):
    nd = len(shape)
    return pl.BlockSpec(shape, lambda i: (0,) * nd)


def _rows(tile, ncols):
    return pl.BlockSpec((tile, ncols), lambda i: (i, 0))


def _split_w(w):
    c = w.shape[0] // 2
    return w[:c], w[c:] - w[:c]


def _ce_stage(s, ix, j, k):
    r, sub, tq = s.shape
    nb = r // (2 * j)
    a = s.reshape(nb, 2, j, sub, tq)
    b = ix.reshape(nb, 2, j, sub, tq)
    dm = (lax.broadcasted_iota(jnp.int32, (nb, 1, 1, 1), 0)
          // (k // (2 * j))) % 2 == 0
    lo, hi = a[:, 0], a[:, 1]
    li, hi_i = b[:, 0], b[:, 1]
    sw = (lo >= hi) != dm
    s_out = jnp.stack([jnp.where(sw, hi, lo), jnp.where(sw, lo, hi)], axis=1)
    i_out = jnp.stack([jnp.where(sw, hi_i, li), jnp.where(sw, li, hi_i)],
                      axis=1)
    return s_out.reshape(r, sub, tq), i_out.reshape(r, sub, tq)


def _select64(s, ix):
    for k in (2, 4, 8, 16, 32, 64):
        j = k // 2
        while j >= 1:
            s, ix = _ce_stage(s, ix, j, k)
            j //= 2
    while s.shape[0] > 64:
        r, sub, tq = s.shape
        a = s.reshape(r // 128, 2, 64, sub, tq)
        b = ix.reshape(r // 128, 2, 64, sub, tq)
        keep = a[:, 0] >= a[:, 1]
        s = jnp.where(keep, a[:, 0], a[:, 1]).reshape(r // 2, sub, tq)
        ix = jnp.where(keep, b[:, 0], b[:, 1]).reshape(r // 2, sub, tq)
        j = 32
        while j >= 1:
            s, ix = _ce_stage(s, ix, j, 64)
            j //= 2
    return s, ix


def _knn_body(x_ref, qt_ref, csq_ref, o_ref, *, nk, n):
    tq = qt_ref.shape[1]
    st = 2.0 * jnp.dot(x_ref[...], qt_ref[...],
                       preferred_element_type=jnp.float32)
    st = st - csq_ref[...]
    sc = st.reshape(n // 8, 8, tq)
    ix = lax.broadcasted_iota(jnp.int32, (n, tq), 0).reshape(n // 8, 8, tq)
    sc, ix = _select64(sc, ix)
    rows = []
    for _ in range(nk):
        m = jnp.max(jnp.max(sc, axis=0), axis=0)[None, None]
        eq = sc == m
        sel = jnp.min(jnp.min(jnp.where(eq, ix, n), axis=0),
                      axis=0)[None, None]
        rows.append(sel.reshape(1, tq))
        sc = jnp.where(eq & (ix == sel), -jnp.inf, sc)
    o_ref[...] = jnp.concatenate(rows, axis=0)


def _knn(x, nk, tq):
    n, c = x.shape
    xt = x.T
    colsq = jnp.sum(x * x, axis=-1, keepdims=True)
    return pl.pallas_call(
        functools.partial(_knn_body, nk=nk, n=n),
        out_shape=jax.ShapeDtypeStruct((nk, n), jnp.int32),
        grid=(n // tq,),
        in_specs=[_rep((n, c)),
                  pl.BlockSpec((c, tq), lambda i: (0, i)),
                  _rep((n, 1))],
        out_specs=pl.BlockSpec((nk, tq), lambda i: (0, i)),
        compiler_params=pltpu.CompilerParams(
            dimension_semantics=("parallel",)),
    )(x, xt, colsq)


def _edge1_body(c_ref, g_ref, w1t, w1d, s1, b1, w2, s2, b2, wnt,
                x_out, y_out, *, nk):
    tn, cin = c_ref.shape
    cw = jnp.dot(c_ref[...], w1d[...],
                 preferred_element_type=jnp.float32)
    g = g_ref[...].reshape(nk * tn, cin)
    h = jnp.dot(g, w1t[...], preferred_element_type=jnp.float32)
    h = h.reshape(nk, tn, 64) + cw[None]
    h = _lrelu(h * s1[...] + b1[...])
    h = jnp.dot(h.reshape(nk * tn, 64), w2[...],
                preferred_element_type=jnp.float32)
    h = _lrelu(h * s2[...] + b2[...]).reshape(nk, tn, 64)
    x = jnp.max(h, axis=0)
    x_out[...] = x
    y_out[...] = jnp.dot(x, wnt[...], preferred_element_type=jnp.float32)


def _edge2_body(c_ref, g_ref, wd, s1, b1, w2, s2, b2, wnt,
                x_out, y_out, *, nk):
    tn = c_ref.shape[0]
    cw = jnp.dot(c_ref[...], wd[...],
                 preferred_element_type=jnp.float32)
    h = _lrelu((g_ref[...] + cw[None]) * s1[...] + b1[...])
    h = jnp.dot(h.reshape(nk * tn, 64), w2[...],
                preferred_element_type=jnp.float32)
    h = _lrelu(h * s2[...] + b2[...]).reshape(nk, tn, 64)
    x = jnp.max(h, axis=0)
    x_out[...] = x
    y_out[...] = jnp.dot(x, wnt[...], preferred_element_type=jnp.float32)


def _edge3_body(c_ref, g_ref, wd, s1, b1, x_out, *, nk):
    cw = jnp.dot(c_ref[...], wd[...],
                 preferred_element_type=jnp.float32)
    h = _lrelu((g_ref[...] + cw[None]) * s1[...] + b1[...])
    x_out[...] = jnp.max(h, axis=0)


def _edge_call(body, centers, gath, consts, n_out, tn, nk):
    n, cin = centers.shape
    cshapes = [w.shape for w in consts]
    out_shape = [jax.ShapeDtypeStruct((n, 64), jnp.float32)
                 for _ in range(n_out)]
    out_specs = [_rows(tn, 64) for _ in range(n_out)]
    return pl.pallas_call(
        functools.partial(body, nk=nk),
        out_shape=out_shape if n_out > 1 else out_shape[0],
        grid=(n // tn,),
        in_specs=[
            _rows(tn, cin),
            pl.BlockSpec((nk, tn, gath.shape[2]), lambda i: (0, i, 0)),
        ] + [_rep(s) for s in cshapes],
        out_specs=out_specs if n_out > 1 else out_specs[0],
        compiler_params=pltpu.CompilerParams(
            dimension_semantics=("parallel",)),
    )(centers, gath, *consts)


def _conv6_body(x_ref, w6, s6, b6, o_ref):
    h = jnp.dot(x_ref[...], w6[...], preferred_element_type=jnp.float32)
    h = _lrelu(h * s6[...] + b6[...])
    o_ref[...] = jnp.max(h, axis=0, keepdims=True)[None]


def _gbias_body(t_ref, w8g, s8, b8, o_ref):
    g = jnp.max(t_ref[...], axis=0, keepdims=True)
    o_ref[...] = (jnp.dot(g, w8g[...], preferred_element_type=jnp.float32)
                  * s8[...] + b8[...])


def _head_body(x_ref, w8, s8, gb8, w9, s9, b9, w10, s10, b10, w11, o_ref):
    h = jnp.dot(x_ref[...], w8[...], preferred_element_type=jnp.float32)
    h = _lrelu(h * s8[...] + gb8[...])
    h = _lrelu(jnp.dot(h, w9[...], preferred_element_type=jnp.float32)
               * s9[...] + b9[...])
    h = _lrelu(jnp.dot(h, w10[...], preferred_element_type=jnp.float32)
               * s10[...] + b10[...])
    o_ref[...] = jnp.dot(h, w11[...], preferred_element_type=jnp.float32)


def kernel(points, conv1_w, conv1_scale, conv1_bias,
           conv2_w, conv2_scale, conv2_bias,
           conv3_w, conv3_scale, conv3_bias,
           conv4_w, conv4_scale, conv4_bias,
           conv5_w, conv5_scale, conv5_bias,
           conv6_w, conv6_scale, conv6_bias,
           conv8_w, conv8_scale, conv8_bias,
           conv9_w, conv9_scale, conv9_bias,
           conv10_w, conv10_scale, conv10_bias,
           conv11_w, conv11_scale, conv11_bias):
    n = points.shape[0]
    nk = 40
    tn = 128 if n % 128 == 0 else n
    tq = 256 if n % 256 == 0 else tn
    tw = 256 if n % 256 == 0 else tn
    r64 = lambda v: v.reshape(1, 64)

    x0 = jnp.pad(points, ((0, 0), (0, 5)))
    w1t, w1d = _split_w(conv1_w)
    w1t = jnp.pad(w1t, ((0, 5), (0, 0)))
    w1d = jnp.pad(w1d, ((0, 5), (0, 0)))
    w3t, w3d = _split_w(conv3_w)
    w5t, w5d = _split_w(conv5_w)

    idx = _knn(x0, nk, tq)
    x1, y2 = _edge_call(
        _edge1_body, x0, x0[idx],
        [w1t, w1d, r64(conv1_scale), r64(conv1_bias),
         conv2_w, r64(conv2_scale), r64(conv2_bias), w3t], 2, tn, nk)

    idx = _knn(x1, nk, tq)
    x2, y3 = _edge_call(
        _edge2_body, x1, y2[idx],
        [w3d, r64(conv3_scale), r64(conv3_bias),
         conv4_w, r64(conv4_scale), r64(conv4_bias), w5t], 2, tn, nk)

    idx = _knn(x2, nk, tq)
    x3 = _edge_call(
        _edge3_body, x2, y3[idx],
        [w5d, r64(conv5_scale), r64(conv5_bias)], 1, tn, nk)

    xcat = jnp.concatenate([x1, x2, x3], axis=1)
    emb = conv6_w.shape[1]
    ntiles = n // tw

    tmax = pl.pallas_call(
        _conv6_body,
        out_shape=jax.ShapeDtypeStruct((ntiles, 1, emb), jnp.float32),
        grid=(ntiles,),
        in_specs=[_rows(tw, 192), _rep((192, emb)),
                  _rep((1, emb)), _rep((1, emb))],
        out_specs=pl.BlockSpec((1, 1, emb), lambda i: (i, 0, 0)),
        compiler_params=pltpu.CompilerParams(
            dimension_semantics=("parallel",)),
    )(xcat, conv6_w, conv6_scale.reshape(1, emb), conv6_bias.reshape(1, emb))

    w8g = conv8_w[:emb]
    s8 = conv8_scale.reshape(1, 256)
    gb8 = pl.pallas_call(
        _gbias_body,
        out_shape=jax.ShapeDtypeStruct((1, 256), jnp.float32),
        grid=(1,),
        in_specs=[_rep((ntiles, emb)), _rep((emb, 256)),
                  _rep((1, 256)), _rep((1, 256))],
        out_specs=_rep((1, 256)),
        compiler_params=pltpu.CompilerParams(
            dimension_semantics=("arbitrary",)),
    )(tmax.reshape(ntiles, emb), w8g, s8, conv8_bias.reshape(1, 256))

    w11p = jnp.pad(conv11_w, ((0, 0), (0, 128 - conv11_w.shape[1])))
    out = pl.pallas_call(
        _head_body,
        out_shape=jax.ShapeDtypeStruct((n, 128), jnp.float32),
        grid=(ntiles,),
        in_specs=[_rows(tw, 192), _rep((192, 256)), _rep((1, 256)),
                  _rep((1, 256)),
                  _rep((256, 256)), _rep((1, 256)), _rep((1, 256)),
                  _rep((256, 128)), _rep((1, 128)), _rep((1, 128)),
                  _rep((128, 128))],
        out_specs=_rows(tw, 128),
        compiler_params=pltpu.CompilerParams(
            dimension_semantics=("parallel",)),
    )(xcat, conv8_w[emb:], s8, gb8,
      conv9_w, conv9_scale.reshape(1, 256), conv9_bias.reshape(1, 256),
      conv10_w, conv10_scale.reshape(1, 128), conv10_bias.reshape(1, 128),
      w11p)
    return out[:, :conv11_w.shape[1]]


# speedup vs baseline: 1.1666x; 1.1666x over previous
import functools

import jax
import jax.numpy as jnp
from jax import lax
from jax.experimental import pallas as pl
from jax.experimental.pallas import tpu as pltpu


def _lrelu(x):
    return jnp.maximum(x, 0.2 * x)


def _rep(shape):
    nd = len(shape)
    return pl.BlockSpec(shape, lambda i: (0,) * nd)


def _rows(tile, ncols):
    return pl.BlockSpec((tile, ncols), lambda i: (i, 0))


def _split_w(w):
    c = w.shape[0] // 2
    return w[:c], w[c:] - w[:c]


def _ce_stage(s, ix, j, k):
    r, sub, tq = s.shape
    nb = r // (2 * j)
    a = s.reshape(nb, 2, j, sub, tq)
    b = ix.reshape(nb, 2, j, sub, tq)
    dm = (lax.broadcasted_iota(jnp.int32, (nb, 1, 1, 1), 0)
          // (k // (2 * j))) % 2 == 0
    lo, hi = a[:, 0], a[:, 1]
    li, hi_i = b[:, 0], b[:, 1]
    sw = (lo >= hi) != dm
    s_out = jnp.stack([jnp.where(sw, hi, lo), jnp.where(sw, lo, hi)], axis=1)
    i_out = jnp.stack([jnp.where(sw, hi_i, li), jnp.where(sw, li, hi_i)],
                      axis=1)
    return s_out.reshape(r, sub, tq), i_out.reshape(r, sub, tq)


def _select64(s, ix):
    for k in (2, 4, 8, 16, 32, 64):
        j = k // 2
        while j >= 1:
            s, ix = _ce_stage(s, ix, j, k)
            j //= 2
    while s.shape[0] > 64:
        r, sub, tq = s.shape
        a = s.reshape(r // 128, 2, 64, sub, tq)
        b = ix.reshape(r // 128, 2, 64, sub, tq)
        keep = a[:, 0] >= a[:, 1]
        s = jnp.where(keep, a[:, 0], a[:, 1]).reshape(r // 2, sub, tq)
        ix = jnp.where(keep, b[:, 0], b[:, 1]).reshape(r // 2, sub, tq)
        j = 32
        while j >= 1:
            s, ix = _ce_stage(s, ix, j, 64)
            j //= 2
    return s, ix


def _knn_body(x_ref, qt_ref, csq_ref, o_ref, *, nk, n):
    tq = qt_ref.shape[1]
    st = 2.0 * jnp.dot(x_ref[...], qt_ref[...],
                       preferred_element_type=jnp.float32)
    st = st - csq_ref[...]
    sc = st.reshape(n // 8, 8, tq)
    ix = lax.broadcasted_iota(jnp.int32, (n, tq), 0).reshape(n // 8, 8, tq)
    sc, ix = _select64(sc, ix)
    rows = []
    for _ in range(nk):
        m = jnp.max(jnp.max(sc, axis=0), axis=0)[None, None]
        eq = sc == m
        sel = jnp.min(jnp.min(jnp.where(eq, ix, n), axis=0),
                      axis=0)[None, None]
        rows.append(sel.reshape(1, tq))
        sc = jnp.where(eq & (ix == sel), -jnp.inf, sc)
    o_ref[...] = jnp.concatenate(rows, axis=0)


def _knn(x, nk, tq):
    n, c = x.shape
    xt = x.T
    colsq = jnp.sum(x * x, axis=-1, keepdims=True)
    return pl.pallas_call(
        functools.partial(_knn_body, nk=nk, n=n),
        out_shape=jax.ShapeDtypeStruct((nk, n), jnp.int32),
        grid=(n // tq,),
        in_specs=[_rep((n, c)),
                  pl.BlockSpec((c, tq), lambda i: (0, i)),
                  _rep((n, 1))],
        out_specs=pl.BlockSpec((nk, tq), lambda i: (0, i)),
        compiler_params=pltpu.CompilerParams(
            dimension_semantics=("parallel",)),
    )(x, xt, colsq)


def _edge1_body(c_ref, g_ref, w1t, w1d, s1, b1, w2, s2, b2, wnt,
                x_out, y_out, *, nk):
    tn, cin = c_ref.shape
    cw = jnp.dot(c_ref[...], w1d[...],
                 preferred_element_type=jnp.float32)
    g = g_ref[...].reshape(nk * tn, cin)
    h = jnp.dot(g, w1t[...], preferred_element_type=jnp.float32)
    h = h.reshape(nk, tn, 64) + cw[None]
    h = _lrelu(h * s1[...] + b1[...])
    h = jnp.dot(h.reshape(nk * tn, 64), w2[...],
                preferred_element_type=jnp.float32)
    h = _lrelu(h * s2[...] + b2[...]).reshape(nk, tn, 64)
    x = jnp.max(h, axis=0)
    x_out[...] = x
    y_out[...] = jnp.dot(x, wnt[...], preferred_element_type=jnp.float32)


def _edge2_body(c_ref, g_ref, wd, s1, b1, w2, s2, b2, wnt,
                x_out, y_out, *, nk):
    tn = c_ref.shape[0]
    cw = jnp.dot(c_ref[...], wd[...],
                 preferred_element_type=jnp.float32)
    h = _lrelu((g_ref[...] + cw[None]) * s1[...] + b1[...])
    h = jnp.dot(h.reshape(nk * tn, 64), w2[...],
                preferred_element_type=jnp.float32)
    h = _lrelu(h * s2[...] + b2[...]).reshape(nk, tn, 64)
    x = jnp.max(h, axis=0)
    x_out[...] = x
    y_out[...] = jnp.dot(x, wnt[...], preferred_element_type=jnp.float32)


def _edge3_body(c_ref, g_ref, wd, s1, b1, x_out, *, nk):
    cw = jnp.dot(c_ref[...], wd[...],
                 preferred_element_type=jnp.float32)
    h = _lrelu((g_ref[...] + cw[None]) * s1[...] + b1[...])
    x_out[...] = jnp.max(h, axis=0)


def _edge_call(body, centers, gath, consts, n_out, tn, nk):
    n, cin = centers.shape
    cshapes = [w.shape for w in consts]
    out_shape = [jax.ShapeDtypeStruct((n, 64), jnp.float32)
                 for _ in range(n_out)]
    out_specs = [_rows(tn, 64) for _ in range(n_out)]
    return pl.pallas_call(
        functools.partial(body, nk=nk),
        out_shape=out_shape if n_out > 1 else out_shape[0],
        grid=(n // tn,),
        in_specs=[
            _rows(tn, cin),
            pl.BlockSpec((nk, tn, gath.shape[2]), lambda i: (0, i, 0)),
        ] + [_rep(s) for s in cshapes],
        out_specs=out_specs if n_out > 1 else out_specs[0],
        compiler_params=pltpu.CompilerParams(
            dimension_semantics=("parallel",)),
    )(centers, gath, *consts)


def _conv6_body(x_ref, w6, s6, b6, o_ref):
    h = jnp.dot(x_ref[...], w6[...], preferred_element_type=jnp.float32)
    h = _lrelu(h * s6[...] + b6[...])
    o_ref[...] = jnp.max(h, axis=0, keepdims=True)[None]


def _gbias_body(t_ref, w8g, s8, b8, o_ref):
    g = jnp.max(t_ref[...], axis=0, keepdims=True)
    o_ref[...] = (jnp.dot(g, w8g[...], preferred_element_type=jnp.float32)
                  * s8[...] + b8[...])


def _head_body(x_ref, w8, s8, gb8, w9, s9, b9, w10, s10, b10, w11, o_ref):
    h = jnp.dot(x_ref[...], w8[...], preferred_element_type=jnp.float32)
    h = _lrelu(h * s8[...] + gb8[...])
    h = _lrelu(jnp.dot(h, w9[...], preferred_element_type=jnp.float32)
               * s9[...] + b9[...])
    h = _lrelu(jnp.dot(h, w10[...], preferred_element_type=jnp.float32)
               * s10[...] + b10[...])
    o_ref[...] = jnp.dot(h, w11[...], preferred_element_type=jnp.float32)


def kernel(points, conv1_w, conv1_scale, conv1_bias,
           conv2_w, conv2_scale, conv2_bias,
           conv3_w, conv3_scale, conv3_bias,
           conv4_w, conv4_scale, conv4_bias,
           conv5_w, conv5_scale, conv5_bias,
           conv6_w, conv6_scale, conv6_bias,
           conv8_w, conv8_scale, conv8_bias,
           conv9_w, conv9_scale, conv9_bias,
           conv10_w, conv10_scale, conv10_bias,
           conv11_w, conv11_scale, conv11_bias):
    n = points.shape[0]
    nk = 40
    tn = 128 if n % 128 == 0 else n
    tq = tn
    tw = 256 if n % 256 == 0 else tn
    r64 = lambda v: v.reshape(1, 64)

    x0 = jnp.pad(points, ((0, 0), (0, 5)))
    w1t, w1d = _split_w(conv1_w)
    w1t = jnp.pad(w1t, ((0, 5), (0, 0)))
    w1d = jnp.pad(w1d, ((0, 5), (0, 0)))
    w3t, w3d = _split_w(conv3_w)
    w5t, w5d = _split_w(conv5_w)

    idx = _knn(x0, nk, tq)
    x1, y2 = _edge_call(
        _edge1_body, x0, x0[idx],
        [w1t, w1d, r64(conv1_scale), r64(conv1_bias),
         conv2_w, r64(conv2_scale), r64(conv2_bias), w3t], 2, tn, nk)

    idx = _knn(x1, nk, tq)
    x2, y3 = _edge_call(
        _edge2_body, x1, y2[idx],
        [w3d, r64(conv3_scale), r64(conv3_bias),
         conv4_w, r64(conv4_scale), r64(conv4_bias), w5t], 2, tn, nk)

    idx = _knn(x2, nk, tq)
    x3 = _edge_call(
        _edge3_body, x2, y3[idx],
        [w5d, r64(conv5_scale), r64(conv5_bias)], 1, tn, nk)

    xcat = jnp.concatenate([x1, x2, x3], axis=1)
    emb = conv6_w.shape[1]
    ntiles = n // tw

    tmax = pl.pallas_call(
        _conv6_body,
        out_shape=jax.ShapeDtypeStruct((ntiles, 1, emb), jnp.float32),
        grid=(ntiles,),
        in_specs=[_rows(tw, 192), _rep((192, emb)),
                  _rep((1, emb)), _rep((1, emb))],
        out_specs=pl.BlockSpec((1, 1, emb), lambda i: (i, 0, 0)),
        compiler_params=pltpu.CompilerParams(
            dimension_semantics=("parallel",)),
    )(xcat, conv6_w, conv6_scale.reshape(1, emb), conv6_bias.reshape(1, emb))

    w8g = conv8_w[:emb]
    s8 = conv8_scale.reshape(1, 256)
    gb8 = pl.pallas_call(
        _gbias_body,
        out_shape=jax.ShapeDtypeStruct((1, 256), jnp.float32),
        grid=(1,),
        in_specs=[_rep((ntiles, emb)), _rep((emb, 256)),
                  _rep((1, 256)), _rep((1, 256))],
        out_specs=_rep((1, 256)),
        compiler_params=pltpu.CompilerParams(
            dimension_semantics=("arbitrary",)),
    )(tmax.reshape(ntiles, emb), w8g, s8, conv8_bias.reshape(1, 256))

    w11p = jnp.pad(conv11_w, ((0, 0), (0, 128 - conv11_w.shape[1])))
    out = pl.pallas_call(
        _head_body,
        out_shape=jax.ShapeDtypeStruct((n, 128), jnp.float32),
        grid=(ntiles,),
        in_specs=[_rows(tw, 192), _rep((192, 256)), _rep((1, 256)),
                  _rep((1, 256)),
                  _rep((256, 256)), _rep((1, 256)), _rep((1, 256)),
                  _rep((256, 128)), _rep((1, 128)), _rep((1, 128)),
                  _rep((128, 128))],
        out_specs=_rows(tw, 128),
        compiler_params=pltpu.CompilerParams(
            dimension_semantics=("parallel",)),
    )(xcat, conv8_w[emb:], s8, gb8,
      conv9_w, conv9_scale.reshape(1, 256), conv9_bias.reshape(1, 256),
      conv10_w, conv10_scale.reshape(1, 128), conv10_bias.reshape(1, 128),
      w11p)
    return out[:, :conv11_w.shape[1]]
